# baseline (speedup 1.0000x reference)
import jax
import jax.numpy as jnp
from jax.experimental import pallas as pl
import jax.experimental.pallas.tpu as pltpu
from jax.experimental.pallas import tpu_sc as plsc

_NUM_IDX = 1000
_PAD = 1024
_RB = 512
_BB = 4
_B = 16
_NW = 32
_EPW = _PAD // _NW


def _quant_sc_kernel(sigma_hbm, sigmas_hbm, po_hbm, sig_v, sv1, out_v):
    c = jax.lax.axis_index("c")
    s = jax.lax.axis_index("s")
    wid = s * 2 + c
    pltpu.sync_copy(sigma_hbm, sig_v)
    pltpu.sync_copy(sigmas_hbm.at[pl.ds(wid * _EPW, _EPW)], sv1)
    sigv = sig_v[...]

    bd = jnp.full((16,), 1e30, jnp.float32)
    bi = jnp.zeros((16,), jnp.float32)
    bv = jnp.zeros((16,), jnp.float32)
    gbase = (wid * _EPW).astype(jnp.float32)
    for g in range(_EPW // _B):
        grp = sv1[pl.ds(g * _B, _B)]
        for j in range(_B):
            e = grp[j]
            d = jnp.abs(sigv - e)
            upd = d < bd
            bi = jnp.where(upd, gbase + float(g * _B + j), bi)
            bv = jnp.where(upd, e, bv)
            bd = jnp.where(upd, d, bd)

    out_v[pl.ds(0, _B)] = bd
    out_v[pl.ds(_B, _B)] = bi
    out_v[pl.ds(2 * _B, _B)] = bv
    pltpu.sync_copy(out_v, po_hbm.at[wid])


def _quantize_sigma(sigma, sigmas_p):
    mesh = plsc.VectorSubcoreMesh(core_axis_name="c", subcore_axis_name="s")
    return pl.kernel(
        _quant_sc_kernel,
        out_type=jax.ShapeDtypeStruct((_NW, 3 * _B), jnp.float32),
        mesh=mesh,
        scratch_types=[
            pltpu.VMEM((_B,), jnp.float32),
            pltpu.VMEM((_EPW,), jnp.float32),
            pltpu.VMEM((3 * _B,), jnp.float32),
        ],
    )(sigma, sigmas_p)


def _dd_kernel(po_ref, w_ref, x_ref, o_ref, a_sm, bias_sm):
    b = pl.program_id(0)

    @pl.when(b == 0)
    def _compute_scalars():
        p = po_ref[:, :]
        d = p[:, 0:_B]
        i = p[:, _B:2 * _B]
        v = p[:, 2 * _B:3 * _B]
        m = jnp.min(d, axis=0, keepdims=True)
        ii = jnp.min(jnp.where(d == m, i, jnp.float32(2 ** 30)),
                     axis=0, keepdims=True)
        sq = jnp.sum(jnp.where((d == m) & (i == ii), v, 0.0),
                     axis=0, keepdims=True)
        a = -sq / jnp.sqrt(sq * sq + 1.0)
        bias = -sq * (ii / _NUM_IDX)
        for k in range(_B):
            a_sm[k] = a[0, k]
            bias_sm[k] = bias[0, k]

    ch = 32
    aws = []
    biases = []
    for bb in range(_BB):
        a_b = a_sm[b * _BB + bb]
        biases.append(bias_sm[b * _BB + bb])
        aws.append([[a_b * w_ref[c, d] + (1.0 if c == d else 0.0)
                     for d in range(3)] for c in range(3)])

    def body(i, carry):
        r = pl.multiple_of(i * ch, ch)
        for bb in range(_BB):
            aw = aws[bb]
            x0 = x_ref[bb, 0, pl.ds(r, ch), :]
            x1 = x_ref[bb, 1, pl.ds(r, ch), :]
            x2 = x_ref[bb, 2, pl.ds(r, ch), :]
            for d in range(3):
                o_ref[bb, d, pl.ds(r, ch), :] = (
                    aw[0][d] * x0 + aw[1][d] * x1 + aw[2][d] * x2 + biases[bb]
                )
        return carry

    jax.lax.fori_loop(0, _RB // ch, body, 0)


@jax.jit
def kernel(inputs, sigma, W, sigmas):
    B, C, H, Wd = inputs.shape
    sigmas_p = jnp.concatenate(
        [sigmas, jnp.full((_PAD - _NUM_IDX,), 1e30, dtype=sigmas.dtype)]
    )
    po = _quantize_sigma(sigma, sigmas_p)
    return pl.pallas_call(
        _dd_kernel,
        grid=(B // _BB,),
        in_specs=[
            pl.BlockSpec((_NW, 3 * _B), lambda b: (0, 0)),
            pl.BlockSpec(memory_space=pltpu.SMEM),
            pl.BlockSpec((_BB, C, _RB, Wd), lambda b: (b, 0, 0, 0)),
        ],
        out_specs=pl.BlockSpec((_BB, C, _RB, Wd), lambda b: (b, 0, 0, 0)),
        out_shape=jax.ShapeDtypeStruct((B, C, H, Wd), inputs.dtype),
        scratch_shapes=[
            pltpu.SMEM((B,), jnp.float32),
            pltpu.SMEM((B,), jnp.float32),
        ],
        compiler_params=pltpu.CompilerParams(
            dimension_semantics=("arbitrary",),
        ),
    )(po, W, inputs)


# speedup vs baseline: 1.0151x; 1.0151x over previous
import jax
import jax.numpy as jnp
from jax.experimental import pallas as pl
import jax.experimental.pallas.tpu as pltpu
from jax.experimental.pallas import tpu_sc as plsc

_NUM_IDX = 1000
_PAD = 1024
_RB = 512
_BB = 4
_B = 16
_NW = 32
_EPW = _PAD // _NW


def _quant_sc_kernel(packed_hbm, po_hbm, pkt_v, out_v):
    c = jax.lax.axis_index("c")
    s = jax.lax.axis_index("s")
    wid = s * 2 + c
    pltpu.sync_copy(packed_hbm.at[wid], pkt_v)
    sigv = pkt_v[pl.ds(0, _B)]

    bd = jnp.full((16,), 1e30, jnp.float32)
    bi = jnp.zeros((16,), jnp.float32)
    bv = jnp.zeros((16,), jnp.float32)
    gbase = (wid * _EPW).astype(jnp.float32)
    for j in range(_EPW):
        row = pkt_v[pl.ds(_B + j * _B, _B)]
        d = jnp.abs(sigv - row)
        upd = d < bd
        bi = jnp.where(upd, gbase + float(j), bi)
        bv = jnp.where(upd, row, bv)
        bd = jnp.where(upd, d, bd)

    out_v[pl.ds(0, _B)] = bd
    out_v[pl.ds(_B, _B)] = bi
    out_v[pl.ds(2 * _B, _B)] = bv
    pltpu.sync_copy(out_v, po_hbm.at[wid])


def _quantize_sigma(packed):
    mesh = plsc.VectorSubcoreMesh(core_axis_name="c", subcore_axis_name="s")
    return pl.kernel(
        _quant_sc_kernel,
        out_type=jax.ShapeDtypeStruct((_NW, 3 * _B), jnp.float32),
        mesh=mesh,
        scratch_types=[
            pltpu.VMEM(((_EPW + 1) * _B,), jnp.float32),
            pltpu.VMEM((3 * _B,), jnp.float32),
        ],
    )(packed)


def _dd_kernel(po_ref, w_ref, x_ref, o_ref, a_sm, bias_sm):
    b = pl.program_id(0)

    @pl.when(b == 0)
    def _compute_scalars():
        p = po_ref[:, :]
        d = p[:, 0:_B]
        i = p[:, _B:2 * _B]
        v = p[:, 2 * _B:3 * _B]
        m = jnp.min(d, axis=0, keepdims=True)
        ii = jnp.min(jnp.where(d == m, i, jnp.float32(2 ** 30)),
                     axis=0, keepdims=True)
        sq = jnp.sum(jnp.where((d == m) & (i == ii), v, 0.0),
                     axis=0, keepdims=True)
        a = -sq / jnp.sqrt(sq * sq + 1.0)
        bias = -sq * (ii / _NUM_IDX)
        for k in range(_B):
            a_sm[k] = a[0, k]
            bias_sm[k] = bias[0, k]

    ch = 32
    aws = []
    biases = []
    for bb in range(_BB):
        a_b = a_sm[b * _BB + bb]
        biases.append(bias_sm[b * _BB + bb])
        aws.append([[a_b * w_ref[c, d] + (1.0 if c == d else 0.0)
                     for d in range(3)] for c in range(3)])

    def body(i, carry):
        r = pl.multiple_of(i * ch, ch)
        for bb in range(_BB):
            aw = aws[bb]
            x0 = x_ref[bb, 0, pl.ds(r, ch), :]
            x1 = x_ref[bb, 1, pl.ds(r, ch), :]
            x2 = x_ref[bb, 2, pl.ds(r, ch), :]
            for d in range(3):
                o_ref[bb, d, pl.ds(r, ch), :] = (
                    aw[0][d] * x0 + aw[1][d] * x1 + aw[2][d] * x2 + biases[bb]
                )
        return carry

    jax.lax.fori_loop(0, _RB // ch, body, 0)


@jax.jit
def kernel(inputs, sigma, W, sigmas):
    B, C, H, Wd = inputs.shape
    sigmas_p = jnp.concatenate(
        [sigmas, jnp.full((_PAD - _NUM_IDX,), 1e30, dtype=sigmas.dtype)]
    )
    rows = jnp.tile(sigmas_p[:, None], (1, _B)).reshape(_NW, _EPW * _B)
    packed = jnp.concatenate([jnp.tile(sigma[None, :], (_NW, 1)), rows], axis=1)
    po = _quantize_sigma(packed)
    return pl.pallas_call(
        _dd_kernel,
        grid=(B // _BB,),
        in_specs=[
            pl.BlockSpec((_NW, 3 * _B), lambda b: (0, 0)),
            pl.BlockSpec(memory_space=pltpu.SMEM),
            pl.BlockSpec((_BB, C, _RB, Wd), lambda b: (b, 0, 0, 0)),
        ],
        out_specs=pl.BlockSpec((_BB, C, _RB, Wd), lambda b: (b, 0, 0, 0)),
        out_shape=jax.ShapeDtypeStruct((B, C, H, Wd), inputs.dtype),
        scratch_shapes=[
            pltpu.SMEM((B,), jnp.float32),
            pltpu.SMEM((B,), jnp.float32),
        ],
        compiler_params=pltpu.CompilerParams(
            dimension_semantics=("arbitrary",),
        ),
    )(po, W, inputs)
